# Initial kernel scaffold; baseline (speedup 1.0000x reference)
import jax
import jax.numpy as jnp
from jax.experimental import pallas as pl


def kernel(dot_p, target):
    raise NotImplementedError("write your pallas kernel here")


# speedup vs baseline: 9.0934x; 9.0934x over previous
import functools

import jax
import jax.numpy as jnp
from jax import lax
from jax.experimental import pallas as pl
from jax.experimental.pallas import tpu as pltpu
from jax.experimental.pallas import tpu_sc as plsc

_B = 4096
_C = 1000
_NC = 2
_NS = 16
_L = 16
_NW = _NC * _NS
_ROWS_PER_W = _B // _NW
_GROUPS = _ROWS_PER_W // _L

_R = 1.0
_DELTA = 0.5

_mesh = plsc.VectorSubcoreMesh(
    core_axis_name="c", subcore_axis_name="s",
    num_cores=_NC, num_subcores=_NS)


@functools.partial(
    pl.kernel,
    out_type=jax.ShapeDtypeStruct((_NW, _L), jnp.float32),
    mesh=_mesh,
    scratch_types=[
        pltpu.VMEM((_L * _C,), jnp.float32),
        pltpu.VMEM((_L * _C,), jnp.float32),
        pltpu.VMEM((_ROWS_PER_W,), jnp.int32),
        pltpu.VMEM((_L,), jnp.float32),
        pltpu.SemaphoreType.DMA,
        pltpu.SemaphoreType.DMA,
    ],
    compiler_params=pltpu.CompilerParams(
        use_tc_tiling_on_sc=False, needs_layout_passes=False),
)
def _npt_loss_sc(dot_hbm, tgt_hbm, out_hbm, buf0, buf1, tgt_v, out_v,
                 sem0, sem1):
    wid = lax.axis_index("s") * _NC + lax.axis_index("c")
    base = wid * _ROWS_PER_W
    pltpu.sync_copy(tgt_hbm.at[pl.ds(base, _ROWS_PER_W)], tgt_v)

    bufs = (buf0, buf1)
    sems = (sem0, sem1)
    copies = [pltpu.async_copy(
        dot_hbm.at[pl.ds(base * _C, _L * _C)], buf0, sem0), None]

    row_iota = lax.iota(jnp.int32, _L)
    zeros = jnp.zeros((_L,), jnp.float32)
    neg_inf = jnp.full((_L,), -jnp.inf, jnp.float32)
    acc = zeros

    for g in range(_GROUPS):
        buf = bufs[g % 2]
        copies[g % 2].wait()
        if g + 1 < _GROUPS:
            copies[(g + 1) % 2] = pltpu.async_copy(
                dot_hbm.at[pl.ds((base + (g + 1) * _L) * _C, _L * _C)],
                bufs[(g + 1) % 2], sems[(g + 1) % 2])

        tgt = tgt_v[pl.ds(g * _L, _L)]
        row_base = row_iota * _C
        tvec = plsc.load_gather(buf, [row_base + tgt])
        plsc.store_scatter(buf, [row_base + tgt], zeros)

        def body(col, carry):
            m1, m2 = carry
            x = plsc.load_gather(buf, [row_base + col])
            m2 = jnp.maximum(m2, jnp.minimum(m1, x))
            m1 = jnp.maximum(m1, x)
            return (m1, m2)

        m1, m2 = lax.fori_loop(0, _C, body, (neg_inf, neg_inf))

        l1 = jnp.maximum(m1 - tvec + _DELTA, 0.0)
        l2 = jnp.maximum(m2 - tvec + _DELTA, 0.0)
        acc = acc + (l1 + l2) * (2.0 * _R)

    out_v[...] = acc
    pltpu.sync_copy(out_v, out_hbm.at[wid])


def kernel(dot_p, target):
    partials = _npt_loss_sc(dot_p.reshape(-1), target.astype(jnp.int32))
    return jnp.sum(partials) / _B


# speedup vs baseline: 12.4137x; 1.3651x over previous
import functools

import jax
import jax.numpy as jnp
from jax import lax
from jax.experimental import pallas as pl
from jax.experimental.pallas import tpu as pltpu
from jax.experimental.pallas import tpu_sc as plsc

_B = 4096
_C = 1000
_NC = 2
_NS = 16
_L = 16
_NW = _NC * _NS
_ROWS_PER_W = _B // _NW
_GROUPS = _ROWS_PER_W // _L

_R = 1.0
_DELTA = 0.5
_UNROLL = 8

_mesh = plsc.VectorSubcoreMesh(
    core_axis_name="c", subcore_axis_name="s",
    num_cores=_NC, num_subcores=_NS)


@functools.partial(
    pl.kernel,
    out_type=jax.ShapeDtypeStruct((_NW, _L), jnp.float32),
    mesh=_mesh,
    scratch_types=[
        pltpu.VMEM((_L * _C,), jnp.float32),
        pltpu.VMEM((_L * _C,), jnp.float32),
        pltpu.VMEM((_ROWS_PER_W,), jnp.int32),
        pltpu.VMEM((_L,), jnp.float32),
        pltpu.SemaphoreType.DMA,
        pltpu.SemaphoreType.DMA,
    ],
    compiler_params=pltpu.CompilerParams(
        use_tc_tiling_on_sc=False, needs_layout_passes=False),
)
def _npt_loss_sc(dot_hbm, tgt_hbm, out_hbm, buf0, buf1, tgt_v, out_v,
                 sem0, sem1):
    wid = lax.axis_index("s") * _NC + lax.axis_index("c")
    base = wid * _ROWS_PER_W
    pltpu.sync_copy(tgt_hbm.at[pl.ds(base, _ROWS_PER_W)], tgt_v)

    bufs = (buf0, buf1)
    sems = (sem0, sem1)
    copies = [pltpu.async_copy(
        dot_hbm.at[pl.ds(base * _C, _L * _C)], buf0, sem0), None]

    row_iota = lax.iota(jnp.int32, _L)
    zeros = jnp.zeros((_L,), jnp.float32)
    neg_inf = jnp.full((_L,), -jnp.inf, jnp.float32)
    acc = zeros

    for g in range(_GROUPS):
        buf = bufs[g % 2]
        copies[g % 2].wait()
        if g + 1 < _GROUPS:
            copies[(g + 1) % 2] = pltpu.async_copy(
                dot_hbm.at[pl.ds((base + (g + 1) * _L) * _C, _L * _C)],
                bufs[(g + 1) % 2], sems[(g + 1) % 2])

        tgt = tgt_v[pl.ds(g * _L, _L)]
        row_base = row_iota * _C
        tvec = plsc.load_gather(buf, [row_base + tgt])
        plsc.store_scatter(buf, [row_base + tgt], zeros)

        def body(blk, carry):
            m1a, m2a, m1b, m2b = carry
            cbase = blk * _UNROLL
            for k in range(0, _UNROLL, 2):
                xa = plsc.load_gather(buf, [row_base + (cbase + k)])
                xb = plsc.load_gather(buf, [row_base + (cbase + k + 1)])
                m2a = jnp.maximum(m2a, jnp.minimum(m1a, xa))
                m1a = jnp.maximum(m1a, xa)
                m2b = jnp.maximum(m2b, jnp.minimum(m1b, xb))
                m1b = jnp.maximum(m1b, xb)
            return (m1a, m2a, m1b, m2b)

        m1a, m2a, m1b, m2b = lax.fori_loop(
            0, _C // _UNROLL, body, (neg_inf, neg_inf, neg_inf, neg_inf))
        m1 = jnp.maximum(m1a, m1b)
        m2 = jnp.maximum(jnp.minimum(m1a, m1b), jnp.maximum(m2a, m2b))

        l1 = jnp.maximum(m1 - tvec + _DELTA, 0.0)
        l2 = jnp.maximum(m2 - tvec + _DELTA, 0.0)
        acc = acc + (l1 + l2) * (2.0 * _R)

    out_v[...] = acc
    pltpu.sync_copy(out_v, out_hbm.at[wid])


def kernel(dot_p, target):
    partials = _npt_loss_sc(dot_p.reshape(-1), target.astype(jnp.int32))
    return jnp.sum(partials) / _B


# speedup vs baseline: 12.4367x; 1.0018x over previous
import functools

import jax
import jax.numpy as jnp
from jax import lax
from jax.experimental import pallas as pl
from jax.experimental.pallas import tpu as pltpu
from jax.experimental.pallas import tpu_sc as plsc

_B = 4096
_C = 1000
_NC = 2
_NS = 16
_L = 16
_NW = _NC * _NS
_ROWS_PER_W = _B // _NW
_GROUPS = _ROWS_PER_W // _L

_R = 1.0
_DELTA = 0.5
_UNROLL = 8

_mesh = plsc.VectorSubcoreMesh(
    core_axis_name="c", subcore_axis_name="s",
    num_cores=_NC, num_subcores=_NS)


@functools.partial(
    pl.kernel,
    out_type=jax.ShapeDtypeStruct((_NW, _L), jnp.float32),
    mesh=_mesh,
    scratch_types=[
        pltpu.VMEM((_L * _C,), jnp.float32),
        pltpu.VMEM((_L * _C,), jnp.float32),
        pltpu.VMEM((_ROWS_PER_W,), jnp.int32),
        pltpu.VMEM((_L,), jnp.float32),
        pltpu.SemaphoreType.DMA,
        pltpu.SemaphoreType.DMA,
    ],
    compiler_params=pltpu.CompilerParams(
        use_tc_tiling_on_sc=False, needs_layout_passes=False),
)
def _npt_loss_sc(dot_hbm, tgt_hbm, out_hbm, buf0, buf1, tgt_v, out_v,
                 sem0, sem1):
    wid = lax.axis_index("s") * _NC + lax.axis_index("c")
    base = wid * _ROWS_PER_W
    pltpu.sync_copy(tgt_hbm.at[pl.ds(base, _ROWS_PER_W)], tgt_v)

    bufs = (buf0, buf1)
    sems = (sem0, sem1)
    copies = [pltpu.async_copy(
        dot_hbm.at[pl.ds(base * _C, _L * _C)], buf0, sem0), None]

    row_iota = lax.iota(jnp.int32, _L)
    zeros = jnp.zeros((_L,), jnp.float32)
    neg_inf = jnp.full((_L,), -jnp.inf, jnp.float32)
    acc = zeros

    for g in range(_GROUPS):
        buf = bufs[g % 2]
        copies[g % 2].wait()
        if g + 1 < _GROUPS:
            copies[(g + 1) % 2] = pltpu.async_copy(
                dot_hbm.at[pl.ds((base + (g + 1) * _L) * _C, _L * _C)],
                bufs[(g + 1) % 2], sems[(g + 1) % 2])

        tgt = tgt_v[pl.ds(g * _L, _L)]
        row_base = row_iota * _C
        tvec = plsc.load_gather(buf, [row_base + tgt])
        plsc.store_scatter(buf, [row_base + tgt], zeros)

        def body(blk, carry):
            idx, m1a, m2a, m1b, m2b = carry
            for k in range(0, _UNROLL, 2):
                xa = plsc.load_gather(buf, [idx])
                xb = plsc.load_gather(buf, [idx + 1])
                idx = idx + 2
                m2a = jnp.maximum(m2a, jnp.minimum(m1a, xa))
                m1a = jnp.maximum(m1a, xa)
                m2b = jnp.maximum(m2b, jnp.minimum(m1b, xb))
                m1b = jnp.maximum(m1b, xb)
            return (idx, m1a, m2a, m1b, m2b)

        _, m1a, m2a, m1b, m2b = lax.fori_loop(
            0, _C // _UNROLL, body,
            (row_base, neg_inf, neg_inf, neg_inf, neg_inf))
        m1 = jnp.maximum(m1a, m1b)
        m2 = jnp.maximum(jnp.minimum(m1a, m1b), jnp.maximum(m2a, m2b))

        l1 = jnp.maximum(m1 - tvec + _DELTA, 0.0)
        l2 = jnp.maximum(m2 - tvec + _DELTA, 0.0)
        acc = acc + (l1 + l2) * (2.0 * _R)

    out_v[...] = acc
    pltpu.sync_copy(out_v, out_hbm.at[wid])


def kernel(dot_p, target):
    partials = _npt_loss_sc(dot_p.reshape(-1), target.astype(jnp.int32))
    return jnp.sum(partials) / _B
